# Initial kernel scaffold; baseline (speedup 1.0000x reference)
import jax
import jax.numpy as jnp
from jax.experimental import pallas as pl


def kernel(input_ids, token_type_ids, word_table, pos_table, type_table, gamma, beta):
    raise NotImplementedError("write your pallas kernel here")


# speedup vs baseline: 1.0260x; 1.0260x over previous
import functools

import jax
import jax.numpy as jnp
from jax import lax
from jax.experimental import pallas as pl
from jax.experimental.pallas import tpu as pltpu
from jax.experimental.pallas import tpu_sc as plsc

_B, _N = 1024, 200
_V, _D = 1000000, 64
_P, _T = 512, 2
_LN_EPS = 1e-12

_NC, _NS, _L = 2, 16, 16
_NW = _NC * _NS
_TT = _B * _N
_C = 128
_CPW = _TT // _NW // _C
_NBUF = 5
_G = _C // _L


def _rsqrt(x):
    i = lax.bitcast_convert_type(x, jnp.int32)
    i = jnp.int32(0x5F3759DF) - lax.shift_right_logical(i, 1)
    y = lax.bitcast_convert_type(i, jnp.float32)
    for _ in range(3):
        y = y * (1.5 - 0.5 * x * y * y)
    return y


@functools.partial(
    pl.kernel,
    mesh=plsc.VectorSubcoreMesh(core_axis_name="c", subcore_axis_name="s"),
    out_type=jax.ShapeDtypeStruct((_TT, _D), jnp.float32),
    compiler_params=pltpu.CompilerParams(
        needs_layout_passes=False, use_tc_tiling_on_sc=False),
    scratch_types=(
        [
            pltpu.VMEM((_CPW, _C), jnp.int32),
            pltpu.VMEM((_CPW, _C), jnp.int32),
            pltpu.VMEM((_T * _N, _D), jnp.float32),
            pltpu.VMEM((_T, _D), jnp.float32),
            pltpu.VMEM((_D,), jnp.float32),
            pltpu.VMEM((_D,), jnp.float32),
        ]
        + [pltpu.VMEM((_C, _D), jnp.float32) for _ in range(2 * _NBUF)]
        + [pltpu.SemaphoreType.DMA for _ in range(2 * _NBUF)]
    ),
)
def _emb_ln_kernel(ids_hbm, cidx_hbm, word_hbm, pos_hbm, type_hbm,
                   gamma_hbm, beta_hbm, out_hbm,
                   ids_v, cidx_v, comb_v, tt_v, gamma_v, beta_v,
                   *bufs_and_sems):
    rows = list(bufs_and_sems[0:_NBUF])
    obufs = list(bufs_and_sems[_NBUF:2 * _NBUF])
    gsems = list(bufs_and_sems[2 * _NBUF:3 * _NBUF])
    osems = list(bufs_and_sems[3 * _NBUF:4 * _NBUF])

    wid = lax.axis_index("s") * _NC + lax.axis_index("c")
    row0 = wid * _CPW

    pltpu.sync_copy(ids_hbm.at[wid], ids_v)
    pltpu.sync_copy(cidx_hbm.at[wid], cidx_v)
    pltpu.sync_copy(gamma_hbm, gamma_v)
    pltpu.sync_copy(beta_hbm, beta_v)
    pltpu.sync_copy(type_hbm, tt_v)
    pltpu.sync_copy(pos_hbm.at[pl.ds(0, _N)], comb_v.at[pl.ds(0, _N)])

    tvregs = [(tt_v[t, pl.ds(16 * j, 16)]) for t in range(_T) for j in range(4)]

    def _build(i, carry):
        n = _N - 1 - i
        for j in range(4):
            p = comb_v[n, pl.ds(16 * j, 16)]
            comb_v[2 * n + 1, pl.ds(16 * j, 16)] = p + tvregs[4 + j]
            comb_v[2 * n, pl.ds(16 * j, 16)] = p + tvregs[j]
        return carry

    lax.fori_loop(0, _N, _build, 0)

    def _fire_gather(c, b):
        pltpu.make_async_copy(word_hbm.at[ids_v.at[c]], rows[b], gsems[b]).start()

    for b in range(_NBUF):
        _fire_gather(b, b)

    gvec = [gamma_v[pl.ds(16 * j, 16)] for j in range(4)]
    bvec = [beta_v[pl.ds(16 * j, 16)] for j in range(4)]

    def _super(g, carry):
        for b in range(_NBUF):
            c = g * _NBUF + b
            pltpu.make_async_copy(word_hbm.at[ids_v.at[c]], rows[b], gsems[b]).wait()

            @pl.when(g >= 1)
            def _():
                pltpu.make_async_copy(
                    obufs[b], out_hbm.at[pl.ds((row0 + c - _NBUF) * _C, _C)],
                    osems[b]).wait()

            rows_b = rows[b]
            obuf_b = obufs[b]

            def _group(gi, carry, c=c, rows_b=rows_b, obuf_b=obuf_b):
                cvec = cidx_v[c, pl.ds(16 * gi, 16)]
                for t in range(16):
                    tl = 16 * gi + t
                    cb = cvec[t]
                    xs = [
                        rows_b[tl, pl.ds(16 * j, 16)]
                        + comb_v[cb, pl.ds(16 * j, 16)]
                        for j in range(4)
                    ]
                    s = (xs[0] + xs[1]) + (xs[2] + xs[3])
                    q = ((xs[0] * xs[0] + xs[1] * xs[1])
                         + (xs[2] * xs[2] + xs[3] * xs[3]))
                    ssum = jnp.sum(s)
                    qsum = jnp.sum(q)
                    mu = ssum * (1.0 / _D)
                    var = qsum * (1.0 / _D) - mu * mu
                    a = _rsqrt(var + _LN_EPS)
                    at = jnp.full((16,), a, jnp.float32)
                    mt = jnp.full((16,), mu * a, jnp.float32)
                    for j in range(4):
                        obuf_b[tl, pl.ds(16 * j, 16)] = \
                            (xs[j] * at - mt) * gvec[j] + bvec[j]
                return carry

            lax.fori_loop(0, _G, _group, 0)

            pltpu.make_async_copy(
                obufs[b], out_hbm.at[pl.ds((row0 + c) * _C, _C)], osems[b]).start()

            @pl.when(c + _NBUF < _CPW)
            def _():
                _fire_gather(c + _NBUF, b)
        return carry

    lax.fori_loop(0, _CPW // _NBUF, _super, 0)

    for b in range(_NBUF):
        pltpu.make_async_copy(
            obufs[b], out_hbm.at[pl.ds((row0 + _CPW - _NBUF + b) * _C, _C)],
            osems[b]).wait()


def kernel(input_ids, token_type_ids, word_table, pos_table, type_table,
           gamma, beta):
    ids = input_ids.reshape(_NW, _CPW, _C)
    cidx = (token_type_ids
            + (jnp.arange(_N, dtype=jnp.int32) * _T)[None, :]).reshape(
                _NW, _CPW, _C)
    out = _emb_ln_kernel(ids, cidx, word_table, pos_table, type_table,
                         gamma, beta)
    return out.reshape(_B, _N, _D)


# speedup vs baseline: 1.2376x; 1.2062x over previous
import functools

import jax
import jax.numpy as jnp
from jax import lax
from jax.experimental import pallas as pl
from jax.experimental.pallas import tpu as pltpu
from jax.experimental.pallas import tpu_sc as plsc

_B, _N = 1024, 200
_V, _D = 1000000, 64
_P, _T = 512, 2
_LN_EPS = 1e-12

_NC, _NS, _L = 2, 16, 16
_NW = _NC * _NS
_TT = _B * _N
_C = 128
_CPW = _TT // _NW // _C
_NBUF = 5
_G = _C // _L


def _rsqrt(x):
    i = lax.bitcast_convert_type(x, jnp.int32)
    i = jnp.int32(0x5F3759DF) - lax.shift_right_logical(i, 1)
    y = lax.bitcast_convert_type(i, jnp.float32)
    for _ in range(3):
        y = y * (1.5 - 0.5 * x * y * y)
    return y


@functools.partial(
    pl.kernel,
    mesh=plsc.VectorSubcoreMesh(core_axis_name="c", subcore_axis_name="s"),
    out_type=jax.ShapeDtypeStruct((_TT, _D), jnp.float32),
    compiler_params=pltpu.CompilerParams(
        needs_layout_passes=False, use_tc_tiling_on_sc=False),
    scratch_types=(
        [
            pltpu.VMEM((_CPW, _C), jnp.int32),
            pltpu.VMEM((_CPW, _C), jnp.int32),
            pltpu.VMEM((_T * _N, _D), jnp.float32),
            pltpu.VMEM((_T, _D), jnp.float32),
            pltpu.VMEM((_D,), jnp.float32),
            pltpu.VMEM((_D,), jnp.float32),
        ]
        + [
            pltpu.VMEM((_L * _L,), jnp.float32),
            pltpu.VMEM((_L * _L,), jnp.float32),
        ]
        + [pltpu.VMEM((_C, _D), jnp.float32) for _ in range(2 * _NBUF)]
        + [pltpu.SemaphoreType.DMA for _ in range(2 * _NBUF)]
    ),
)
def _emb_ln_kernel(ids_hbm, cidx_hbm, word_hbm, pos_hbm, type_hbm,
                   gamma_hbm, beta_hbm, out_hbm,
                   ids_v, cidx_v, comb_v, tt_v, gamma_v, beta_v,
                   st_v, qt_v,
                   *bufs_and_sems):
    rows = list(bufs_and_sems[0:_NBUF])
    obufs = list(bufs_and_sems[_NBUF:2 * _NBUF])
    gsems = list(bufs_and_sems[2 * _NBUF:3 * _NBUF])
    osems = list(bufs_and_sems[3 * _NBUF:4 * _NBUF])

    wid = lax.axis_index("s") * _NC + lax.axis_index("c")
    row0 = wid * _CPW

    pltpu.sync_copy(ids_hbm.at[wid], ids_v)
    pltpu.sync_copy(cidx_hbm.at[wid], cidx_v)
    pltpu.sync_copy(gamma_hbm, gamma_v)
    pltpu.sync_copy(beta_hbm, beta_v)
    pltpu.sync_copy(type_hbm, tt_v)
    pltpu.sync_copy(pos_hbm.at[pl.ds(0, _N)], comb_v.at[pl.ds(0, _N)])

    tvregs = [(tt_v[t, pl.ds(16 * j, 16)]) for t in range(_T) for j in range(4)]

    def _build(i, carry):
        n = _N - 1 - i
        for j in range(4):
            p = comb_v[n, pl.ds(16 * j, 16)]
            comb_v[2 * n + 1, pl.ds(16 * j, 16)] = p + tvregs[4 + j]
            comb_v[2 * n, pl.ds(16 * j, 16)] = p + tvregs[j]
        return carry

    lax.fori_loop(0, _N, _build, 0)

    def _fire_gather(c, b):
        pltpu.make_async_copy(word_hbm.at[ids_v.at[c]], rows[b], gsems[b]).start()

    for b in range(_NBUF):
        _fire_gather(b, b)

    gvec = [gamma_v[pl.ds(16 * j, 16)] for j in range(4)]
    bvec = [beta_v[pl.ds(16 * j, 16)] for j in range(4)]
    iota16 = lax.iota(jnp.int32, 16)

    def _super(g, carry):
        for b in range(_NBUF):
            c = g * _NBUF + b
            pltpu.make_async_copy(word_hbm.at[ids_v.at[c]], rows[b], gsems[b]).wait()

            @pl.when(g >= 1)
            def _():
                pltpu.make_async_copy(
                    obufs[b], out_hbm.at[pl.ds((row0 + c - _NBUF) * _C, _C)],
                    osems[b]).wait()

            rows_b = rows[b]
            obuf_b = obufs[b]

            def _group(gi, carry, c=c, rows_b=rows_b, obuf_b=obuf_b):
                cvec = cidx_v[c, pl.ds(16 * gi, 16)]
                for t in range(16):
                    tl = 16 * gi + t
                    cb = cvec[t]
                    xs = [
                        rows_b[tl, pl.ds(16 * j, 16)]
                        + comb_v[cb, pl.ds(16 * j, 16)]
                        for j in range(4)
                    ]
                    s = (xs[0] + xs[1]) + (xs[2] + xs[3])
                    q = ((xs[0] * xs[0] + xs[1] * xs[1])
                         + (xs[2] * xs[2] + xs[3] * xs[3]))
                    for j in range(4):
                        obuf_b[tl, pl.ds(16 * j, 16)] = xs[j]
                    st_v[pl.ds(16 * t, 16)] = s
                    qt_v[pl.ds(16 * t, 16)] = q
                s = plsc.load_gather(st_v, [iota16 * 16])
                q = plsc.load_gather(qt_v, [iota16 * 16])
                for l in range(1, 16):
                    s = s + plsc.load_gather(st_v, [iota16 * 16 + l])
                    q = q + plsc.load_gather(qt_v, [iota16 * 16 + l])
                mu = s * (1.0 / _D)
                var = q * (1.0 / _D) - mu * mu
                a = _rsqrt(var + _LN_EPS)
                ma = mu * a
                for t in range(16):
                    tl = 16 * gi + t
                    at = jnp.full((16,), a[t], jnp.float32)
                    mt = jnp.full((16,), ma[t], jnp.float32)
                    for j in range(4):
                        x = obuf_b[tl, pl.ds(16 * j, 16)]
                        obuf_b[tl, pl.ds(16 * j, 16)] = \
                            (x * at - mt) * gvec[j] + bvec[j]
                return carry

            lax.fori_loop(0, _G, _group, 0)

            pltpu.make_async_copy(
                obufs[b], out_hbm.at[pl.ds((row0 + c) * _C, _C)], osems[b]).start()

            @pl.when(c + _NBUF < _CPW)
            def _():
                _fire_gather(c + _NBUF, b)
        return carry

    lax.fori_loop(0, _CPW // _NBUF, _super, 0)

    for b in range(_NBUF):
        pltpu.make_async_copy(
            obufs[b], out_hbm.at[pl.ds((row0 + _CPW - _NBUF + b) * _C, _C)],
            osems[b]).wait()


def kernel(input_ids, token_type_ids, word_table, pos_table, type_table,
           gamma, beta):
    ids = input_ids.reshape(_NW, _CPW, _C)
    cidx = (token_type_ids
            + (jnp.arange(_N, dtype=jnp.int32) * _T)[None, :]).reshape(
                _NW, _CPW, _C)
    out = _emb_ln_kernel(ids, cidx, word_table, pos_table, type_table,
                         gamma, beta)
    return out.reshape(_B, _N, _D)


# speedup vs baseline: 1.2392x; 1.0013x over previous
import functools

import jax
import jax.numpy as jnp
from jax import lax
from jax.experimental import pallas as pl
from jax.experimental.pallas import tpu as pltpu
from jax.experimental.pallas import tpu_sc as plsc

_B, _N = 1024, 200
_V, _D = 1000000, 64
_P, _T = 512, 2
_LN_EPS = 1e-12

_NC, _NS, _L = 2, 16, 16
_NW = _NC * _NS
_BPW = _B // _NW
_NBUF = 2
_G16 = _N // 16
_REM = _N - 16 * _G16


def _rsqrt(x):
    i = lax.bitcast_convert_type(x, jnp.int32)
    i = jnp.int32(0x5F3759DF) - lax.shift_right_logical(i, 1)
    y = lax.bitcast_convert_type(i, jnp.float32)
    for _ in range(3):
        y = y * (1.5 - 0.5 * x * y * y)
    return y


@functools.partial(
    pl.kernel,
    mesh=plsc.VectorSubcoreMesh(core_axis_name="c", subcore_axis_name="s"),
    out_type=jax.ShapeDtypeStruct((_B, _N, _D), jnp.float32),
    compiler_params=pltpu.CompilerParams(
        needs_layout_passes=False, use_tc_tiling_on_sc=False),
    scratch_types=(
        [
            pltpu.VMEM((_BPW, _N), jnp.int32),
            pltpu.VMEM((_BPW, _N), jnp.int32),
            pltpu.VMEM((_T * _N, _D), jnp.float32),
            pltpu.VMEM((_T, _D), jnp.float32),
            pltpu.VMEM((_D,), jnp.float32),
            pltpu.VMEM((_D,), jnp.float32),
            pltpu.VMEM((_L * _L,), jnp.float32),
            pltpu.VMEM((_L * _L,), jnp.float32),
        ]
        + [pltpu.VMEM((_N, _D), jnp.float32) for _ in range(2 * _NBUF)]
        + [pltpu.SemaphoreType.DMA for _ in range(2 * _NBUF)]
    ),
)
def _emb_ln_kernel(ids_hbm, cidx_hbm, word_hbm, pos_hbm, type_hbm,
                   gamma_hbm, beta_hbm, out_hbm,
                   ids_v, cidx_v, comb_v, tt_v, gamma_v, beta_v, st_v, qt_v,
                   *bufs_and_sems):
    rows = list(bufs_and_sems[0:_NBUF])
    obufs = list(bufs_and_sems[_NBUF:2 * _NBUF])
    gsems = list(bufs_and_sems[2 * _NBUF:3 * _NBUF])
    osems = list(bufs_and_sems[3 * _NBUF:4 * _NBUF])

    wid = lax.axis_index("s") * _NC + lax.axis_index("c")
    brow0 = wid * _BPW

    pltpu.sync_copy(ids_hbm.at[pl.ds(brow0, _BPW)], ids_v)
    pltpu.sync_copy(cidx_hbm.at[pl.ds(brow0, _BPW)], cidx_v)
    pltpu.sync_copy(gamma_hbm, gamma_v)
    pltpu.sync_copy(beta_hbm, beta_v)
    pltpu.sync_copy(type_hbm, tt_v)
    pltpu.sync_copy(pos_hbm.at[pl.ds(0, _N)], comb_v.at[pl.ds(0, _N)])

    tvregs = [(tt_v[t, pl.ds(16 * j, 16)]) for t in range(_T) for j in range(4)]

    def _build(i, carry):
        n = _N - 1 - i
        for j in range(4):
            p = comb_v[n, pl.ds(16 * j, 16)]
            comb_v[2 * n + 1, pl.ds(16 * j, 16)] = p + tvregs[4 + j]
            comb_v[2 * n, pl.ds(16 * j, 16)] = p + tvregs[j]
        return carry

    lax.fori_loop(0, _N, _build, 0)

    def _fire_gather(kb, b):
        pltpu.make_async_copy(word_hbm.at[ids_v.at[kb]], rows[b], gsems[b]).start()

    for b in range(_NBUF):
        _fire_gather(b, b)

    gvec = [gamma_v[pl.ds(16 * j, 16)] for j in range(4)]
    bvec = [beta_v[pl.ds(16 * j, 16)] for j in range(4)]
    iota16 = lax.iota(jnp.int32, 16)

    def _super(g, carry):
        for b in range(_NBUF):
            kb = g * _NBUF + b
            pltpu.make_async_copy(word_hbm.at[ids_v.at[kb]], rows[b], gsems[b]).wait()

            @pl.when(g >= 1)
            def _():
                pltpu.make_async_copy(
                    obufs[b], out_hbm.at[brow0 + kb - _NBUF], osems[b]).wait()

            rows_b = rows[b]
            obuf_b = obufs[b]

            def _tok(tl, cb, rows_b, obuf_b):
                xs = [
                    rows_b[tl, pl.ds(16 * j, 16)]
                    + comb_v[cb, pl.ds(16 * j, 16)]
                    for j in range(4)
                ]
                s = (xs[0] + xs[1]) + (xs[2] + xs[3])
                q = ((xs[0] * xs[0] + xs[1] * xs[1])
                     + (xs[2] * xs[2] + xs[3] * xs[3]))
                for j in range(4):
                    obuf_b[tl, pl.ds(16 * j, 16)] = xs[j]
                return s, q

            def _stats():
                s = plsc.load_gather(st_v, [iota16 * 16])
                q = plsc.load_gather(qt_v, [iota16 * 16])
                for l in range(1, 16):
                    s = s + plsc.load_gather(st_v, [iota16 * 16 + l])
                    q = q + plsc.load_gather(qt_v, [iota16 * 16 + l])
                mu = s * (1.0 / _D)
                var = q * (1.0 / _D) - mu * mu
                a = _rsqrt(var + _LN_EPS)
                return a, mu * a

            def _norm(tl, a_s, ma_s, obuf_b):
                at = jnp.full((16,), a_s, jnp.float32)
                mt = jnp.full((16,), ma_s, jnp.float32)
                for j in range(4):
                    x = obuf_b[tl, pl.ds(16 * j, 16)]
                    obuf_b[tl, pl.ds(16 * j, 16)] = \
                        (x * at - mt) * gvec[j] + bvec[j]

            def _group(gi, carry, kb=kb, rows_b=rows_b, obuf_b=obuf_b):
                cvec = cidx_v[kb, pl.ds(16 * gi, 16)]
                for t in range(16):
                    s, q = _tok(16 * gi + t, cvec[t], rows_b, obuf_b)
                    st_v[pl.ds(16 * t, 16)] = s
                    qt_v[pl.ds(16 * t, 16)] = q
                a, ma = _stats()
                for t in range(16):
                    _norm(16 * gi + t, a[t], ma[t], obuf_b)
                return carry

            lax.fori_loop(0, _G16, _group, 0)

            cvec = cidx_v[kb, pl.ds(_N - 16, 16)]
            for t in range(_REM):
                s, q = _tok(16 * _G16 + t, cvec[8 + t], rows_b, obuf_b)
                st_v[pl.ds(16 * t, 16)] = s
                qt_v[pl.ds(16 * t, 16)] = q
            a, ma = _stats()
            for t in range(_REM):
                _norm(16 * _G16 + t, a[t], ma[t], obuf_b)

            pltpu.make_async_copy(
                obufs[b], out_hbm.at[brow0 + kb], osems[b]).start()

            @pl.when(kb + _NBUF < _BPW)
            def _():
                _fire_gather(kb + _NBUF, b)
        return carry

    lax.fori_loop(0, _BPW // _NBUF, _super, 0)

    for b in range(_NBUF):
        pltpu.make_async_copy(
            obufs[b], out_hbm.at[brow0 + _BPW - _NBUF + b], osems[b]).wait()


def kernel(input_ids, token_type_ids, word_table, pos_table, type_table,
           gamma, beta):
    cidx = token_type_ids + (jnp.arange(_N, dtype=jnp.int32) * _T)[None, :]
    return _emb_ln_kernel(input_ids, cidx, word_table, pos_table, type_table,
                          gamma, beta)


# speedup vs baseline: 1.2396x; 1.0004x over previous
import functools

import jax
import jax.numpy as jnp
from jax import lax
from jax.experimental import pallas as pl
from jax.experimental.pallas import tpu as pltpu
from jax.experimental.pallas import tpu_sc as plsc

_B, _N = 1024, 200
_V, _D = 1000000, 64
_P, _T = 512, 2
_LN_EPS = 1e-12

_NC, _NS, _L = 2, 16, 16
_NW = _NC * _NS
_BPW = _B // _NW
_NBUF = 2
_G16 = _N // 16
_REM = _N - 16 * _G16


def _rsqrt(x):
    i = lax.bitcast_convert_type(x, jnp.int32)
    i = jnp.int32(0x5F3759DF) - lax.shift_right_logical(i, 1)
    y = lax.bitcast_convert_type(i, jnp.float32)
    for _ in range(3):
        y = y * (1.5 - 0.5 * x * y * y)
    return y


@functools.partial(
    pl.kernel,
    mesh=plsc.VectorSubcoreMesh(core_axis_name="c", subcore_axis_name="s"),
    out_type=jax.ShapeDtypeStruct((_B * _N // 2, 2 * _D), jnp.float32),
    compiler_params=pltpu.CompilerParams(
        needs_layout_passes=False, use_tc_tiling_on_sc=False),
    scratch_types=(
        [
            pltpu.VMEM((_BPW, _N), jnp.int32),
            pltpu.VMEM((_BPW, _N), jnp.int32),
            pltpu.VMEM((_T * _N, _D), jnp.float32),
            pltpu.VMEM((_T, _D), jnp.float32),
            pltpu.VMEM((_D,), jnp.float32),
            pltpu.VMEM((_D,), jnp.float32),
            pltpu.VMEM((_L * _L,), jnp.float32),
            pltpu.VMEM((_L * _L,), jnp.float32),
        ]
        + [pltpu.VMEM((_N, _D), jnp.float32) for _ in range(_NBUF)]
        + [pltpu.VMEM((_N // 2, 2 * _D), jnp.float32) for _ in range(_NBUF)]
        + [pltpu.SemaphoreType.DMA for _ in range(2 * _NBUF)]
    ),
)
def _emb_ln_kernel(ids_hbm, cidx_hbm, word_hbm, pos_hbm, type_hbm,
                   gamma_hbm, beta_hbm, out_hbm,
                   ids_v, cidx_v, comb_v, tt_v, gamma_v, beta_v, st_v, qt_v,
                   *bufs_and_sems):
    rows = list(bufs_and_sems[0:_NBUF])
    obufs = list(bufs_and_sems[_NBUF:2 * _NBUF])
    gsems = list(bufs_and_sems[2 * _NBUF:3 * _NBUF])
    osems = list(bufs_and_sems[3 * _NBUF:4 * _NBUF])

    wid = lax.axis_index("s") * _NC + lax.axis_index("c")
    brow0 = wid * _BPW

    pltpu.sync_copy(ids_hbm.at[pl.ds(brow0, _BPW)], ids_v)
    pltpu.sync_copy(cidx_hbm.at[pl.ds(brow0, _BPW)], cidx_v)
    pltpu.sync_copy(gamma_hbm, gamma_v)
    pltpu.sync_copy(beta_hbm, beta_v)
    pltpu.sync_copy(type_hbm, tt_v)
    pltpu.sync_copy(pos_hbm.at[pl.ds(0, _N)], comb_v.at[pl.ds(0, _N)])

    tvregs = [(tt_v[t, pl.ds(16 * j, 16)]) for t in range(_T) for j in range(4)]

    def _build(i, carry):
        n = _N - 1 - i
        for j in range(4):
            p = comb_v[n, pl.ds(16 * j, 16)]
            comb_v[2 * n + 1, pl.ds(16 * j, 16)] = p + tvregs[4 + j]
            comb_v[2 * n, pl.ds(16 * j, 16)] = p + tvregs[j]
        return carry

    lax.fori_loop(0, _N, _build, 0)

    def _fire_gather(kb, b):
        pltpu.make_async_copy(word_hbm.at[ids_v.at[kb]], rows[b], gsems[b]).start()

    for b in range(_NBUF):
        _fire_gather(b, b)

    gvec = [gamma_v[pl.ds(16 * j, 16)] for j in range(4)]
    bvec = [beta_v[pl.ds(16 * j, 16)] for j in range(4)]
    iota16 = lax.iota(jnp.int32, 16)

    def _super(g, carry):
        for b in range(_NBUF):
            kb = g * _NBUF + b
            pltpu.make_async_copy(word_hbm.at[ids_v.at[kb]], rows[b], gsems[b]).wait()

            @pl.when(g >= 1)
            def _():
                pltpu.make_async_copy(
                    obufs[b],
                    out_hbm.at[pl.ds((brow0 + kb - _NBUF) * (_N // 2), _N // 2)],
                    osems[b]).wait()

            rows_b = rows[b]
            obuf_b = obufs[b]

            def _tok(tl, orow, ocol, cb, rows_b, obuf_b):
                xs = [
                    rows_b[tl, pl.ds(16 * j, 16)]
                    + comb_v[cb, pl.ds(16 * j, 16)]
                    for j in range(4)
                ]
                s = (xs[0] + xs[1]) + (xs[2] + xs[3])
                q = ((xs[0] * xs[0] + xs[1] * xs[1])
                     + (xs[2] * xs[2] + xs[3] * xs[3]))
                for j in range(4):
                    obuf_b[orow, pl.ds(ocol + 16 * j, 16)] = xs[j]
                return s, q

            def _stats():
                s = plsc.load_gather(st_v, [iota16 * 16])
                q = plsc.load_gather(qt_v, [iota16 * 16])
                for l in range(1, 16):
                    s = s + plsc.load_gather(st_v, [iota16 * 16 + l])
                    q = q + plsc.load_gather(qt_v, [iota16 * 16 + l])
                mu = s * (1.0 / _D)
                var = q * (1.0 / _D) - mu * mu
                a = _rsqrt(var + _LN_EPS)
                return a, mu * a

            def _norm(orow, ocol, a_s, ma_s, obuf_b):
                at = jnp.full((16,), a_s, jnp.float32)
                mt = jnp.full((16,), ma_s, jnp.float32)
                for j in range(4):
                    x = obuf_b[orow, pl.ds(ocol + 16 * j, 16)]
                    obuf_b[orow, pl.ds(ocol + 16 * j, 16)] = \
                        (x * at - mt) * gvec[j] + bvec[j]

            def _group(gi, carry, kb=kb, rows_b=rows_b, obuf_b=obuf_b):
                cvec = cidx_v[kb, pl.ds(16 * gi, 16)]
                for t in range(16):
                    s, q = _tok(16 * gi + t, 8 * gi + t // 2, (t % 2) * _D,
                                cvec[t], rows_b, obuf_b)
                    st_v[pl.ds(16 * t, 16)] = s
                    qt_v[pl.ds(16 * t, 16)] = q
                a, ma = _stats()
                for t in range(16):
                    _norm(8 * gi + t // 2, (t % 2) * _D, a[t], ma[t], obuf_b)
                return carry

            lax.fori_loop(0, _G16, _group, 0)

            cvec = cidx_v[kb, pl.ds(_N - 16, 16)]
            for t in range(_REM):
                s, q = _tok(16 * _G16 + t, 8 * _G16 + t // 2, (t % 2) * _D,
                            cvec[8 + t], rows_b, obuf_b)
                st_v[pl.ds(16 * t, 16)] = s
                qt_v[pl.ds(16 * t, 16)] = q
            a, ma = _stats()
            for t in range(_REM):
                _norm(8 * _G16 + t // 2, (t % 2) * _D, a[t], ma[t], obuf_b)

            pltpu.make_async_copy(
                obufs[b],
                out_hbm.at[pl.ds((brow0 + kb) * (_N // 2), _N // 2)],
                osems[b]).start()

            @pl.when(kb + _NBUF < _BPW)
            def _():
                _fire_gather(kb + _NBUF, b)
        return carry

    lax.fori_loop(0, _BPW // _NBUF, _super, 0)

    for b in range(_NBUF):
        pltpu.make_async_copy(
            obufs[b],
            out_hbm.at[pl.ds((brow0 + _BPW - _NBUF + b) * (_N // 2), _N // 2)],
            osems[b]).wait()


def kernel(input_ids, token_type_ids, word_table, pos_table, type_table,
           gamma, beta):
    cidx = token_type_ids + (jnp.arange(_N, dtype=jnp.int32) * _T)[None, :]
    out = _emb_ln_kernel(input_ids, cidx, word_table, pos_table, type_table,
                         gamma, beta)
    return out.reshape(_B, _N, _D)
